# baseline (speedup 1.0000x reference)
import functools

import jax
import jax.numpy as jnp
from jax import lax
from jax.experimental import pallas as pl
from jax.experimental.pallas import tpu as pltpu
from jax.experimental.pallas import tpu_sc as plsc

N = 10000
E = 160000
D = 128
G = 16
NSETS = 4
NC = 2
NS = 16
NW = NC * NS
CH = 128

E2 = 2 * E
NSPLIT = 2
ES = E // NSPLIT
ZROWS = 80
ZCHUNKS = N // ZROWS

DCH = 1280
DCHUNKS = NSETS * (E // DCH)
DPER = -(-DCHUNKS // NW)
DSUB = DCH // 16

BE = 2000
NBE = E // BE
BN = 2000
NBN = N // BN

_f32 = jnp.float32
_i32 = jnp.int32


def _dot(a, b):
    return jax.lax.dot_general(
        a, b, (((1,), (0,)), ((), ())),
        precision=jax.lax.Precision.HIGHEST,
        preferred_element_type=_f32)


def _ln(x, g, b):
    mu = jnp.mean(x, axis=-1, keepdims=True)
    var = jnp.mean((x - mu) * (x - mu), axis=-1, keepdims=True)
    return (x - mu) * jax.lax.rsqrt(var + 1e-5) * g + b


def _silu(x):
    return x * jax.nn.sigmoid(x)


def _sc_mesh():
    return plsc.VectorSubcoreMesh(core_axis_name="c", subcore_axis_name="s")


def _nper(nchunks):
    n = -(-nchunks // NW)
    return n + (n % 2)


@functools.lru_cache(maxsize=None)
def _dist2_call():
    def body(px_hbm, py_hbm, pz_hbm, idx_hbm, out_hbm,
             px_v, py_v, pz_v, idxd_v, idxs_v, out_v):
        cid = lax.axis_index("c")
        sid = lax.axis_index("s")
        wid = sid * NC + cid

        pltpu.sync_copy(px_hbm, px_v)
        pltpu.sync_copy(py_hbm, py_v)
        pltpu.sync_copy(pz_hbm, pz_v)

        def step(j, carry):
            chunk = wid * DPER + j

            @pl.when(chunk < DCHUNKS)
            def _():
                s = chunk // (E // DCH)
                e0 = (chunk % (E // DCH)) * DCH
                pltpu.sync_copy(idx_hbm.at[pl.ds(s * E2 + e0, DCH)], idxd_v)
                pltpu.sync_copy(idx_hbm.at[pl.ds(s * E2 + E + e0, DCH)], idxs_v)

                def grp(k, c2):
                    o = k * 16
                    gd = idxd_v[pl.ds(o, 16)]
                    gs = idxs_v[pl.ds(o, 16)]
                    dx = (plsc.load_gather(px_v, [gd])
                          - plsc.load_gather(px_v, [gs]))
                    dy = (plsc.load_gather(py_v, [gd])
                          - plsc.load_gather(py_v, [gs]))
                    dz = (plsc.load_gather(pz_v, [gd])
                          - plsc.load_gather(pz_v, [gs]))
                    out_v[pl.ds(o, 16)] = dx * dx + dy * dy + dz * dz
                    return c2
                lax.fori_loop(0, DSUB, grp, 0)
                pltpu.sync_copy(out_v, out_hbm.at[pl.ds(chunk * DCH, DCH)])
            return carry

        lax.fori_loop(0, DPER, step, 0)

    return pl.kernel(
        body,
        out_type=jax.ShapeDtypeStruct((NSETS * E,), _f32),
        mesh=_sc_mesh(),
        scratch_types=[
            pltpu.VMEM((N,), _f32),
            pltpu.VMEM((N,), _f32),
            pltpu.VMEM((N,), _f32),
            pltpu.VMEM((DCH,), _i32),
            pltpu.VMEM((DCH,), _i32),
            pltpu.VMEM((DCH,), _f32),
        ],
        compiler_params=pltpu.CompilerParams(needs_layout_passes=False),
    )


@functools.lru_cache(maxsize=None)
def _gather_call(nidx):
    nchunks = nidx // CH
    nper = _nper(nchunks)

    def body(tbl_hbm, idx_hbm, out_hbm, idx0_v, idx1_v, rows_v,
             semg, semw0, semw1, semi0, semi1):
        cid = lax.axis_index("c")
        sid = lax.axis_index("s")
        wid = sid * NC + cid

        for b, idxv in ((0, idx0_v), (1, idx1_v)):
            chunk0 = wid * nper + b

            @pl.when(chunk0 < nchunks)
            def _():
                pltpu.sync_copy(idx_hbm.at[pl.ds(chunk0 * CH, CH)], idxv)

        def sub(j, b, semw, semi, idxv):
            chunk = wid * nper + j
            buf = rows_v.at[pl.ds(b * CH, CH)]

            @pl.when(chunk < nchunks)
            def _():
                @pl.when(j >= 2)
                def _():
                    pltpu.make_async_copy(
                        idx_hbm.at[pl.ds(chunk * CH, CH)], idxv, semi).wait()
                    pltpu.make_async_copy(
                        buf, out_hbm.at[pl.ds(chunk * CH, CH)], semw).wait()
                pltpu.async_copy(tbl_hbm.at[idxv], buf, semg).wait()
                pltpu.async_copy(buf, out_hbm.at[pl.ds(chunk * CH, CH)], semw)

                @pl.when((j + 2 < nper) & (chunk + 2 < nchunks))
                def _():
                    pltpu.async_copy(
                        idx_hbm.at[pl.ds((chunk + 2) * CH, CH)], idxv, semi)

        def step(j2, carry):
            sub(2 * j2, 0, semw0, semi0, idx0_v)
            sub(2 * j2 + 1, 1, semw1, semi1, idx1_v)
            return carry

        lax.fori_loop(0, nper // 2, step, 0)

        nvalid = jnp.maximum(jnp.minimum(nper, nchunks - wid * nper), 0)
        for b, semw in ((0, semw0), (1, semw1)):
            @pl.when(nvalid >= b + 1)
            def _():
                pltpu.make_async_copy(
                    rows_v.at[pl.ds(b * CH, CH)],
                    out_hbm.at[pl.ds(wid * nper * CH, CH)], semw).wait()

    return pl.kernel(
        body,
        out_type=jax.ShapeDtypeStruct((nidx, D), _f32),
        mesh=_sc_mesh(),
        scratch_types=[
            pltpu.VMEM((CH,), _i32),
            pltpu.VMEM((CH,), _i32),
            pltpu.VMEM((2 * CH, D), _f32),
            pltpu.SemaphoreType.DMA,
            pltpu.SemaphoreType.DMA,
            pltpu.SemaphoreType.DMA,
            pltpu.SemaphoreType.DMA,
            pltpu.SemaphoreType.DMA,
        ],
    )


@functools.lru_cache(maxsize=None)
def _scatter_call(ne):
    nchunks = ne // CH
    nper = _nper(nchunks)
    nz = -(-ZCHUNKS // NS)

    def body(m_hbm, dst_hbm, out_hbm, acc, zbuf, idx0_v, idx1_v, rows_v,
             semz, semm0, semm1, semi0, semi1):
        cid = lax.axis_index("c")
        sid = lax.axis_index("s")
        wid = sid * NC + cid

        for b, idxv in ((0, idx0_v), (1, idx1_v)):
            chunk0 = wid * nper + b

            @pl.when(chunk0 < nchunks)
            def _():
                pltpu.sync_copy(dst_hbm.at[pl.ds(chunk0 * CH, CH)], idxv)

        def zfill(i, carry):
            for j in range(8):
                zbuf[i, pl.ds(j * 16, 16)] = jnp.zeros((16,), _f32)
            return carry
        lax.fori_loop(0, ZROWS, zfill, 0)

        def zero_fire(t, carry):
            j = sid + NS * t

            @pl.when(j < ZCHUNKS)
            def _():
                pltpu.async_copy(zbuf, acc.at[pl.ds(j * ZROWS, ZROWS)], semz)
            return carry
        lax.fori_loop(0, nz, zero_fire, 0)

        def zero_drain(t, carry):
            j = sid + NS * t

            @pl.when(j < ZCHUNKS)
            def _():
                pltpu.make_async_copy(
                    zbuf, acc.at[pl.ds(0, ZROWS)], semz).wait()
            return carry
        lax.fori_loop(0, nz, zero_drain, 0)
        plsc.subcore_barrier()

        for b, semm in ((0, semm0), (1, semm1)):
            chunk0 = wid * nper + b

            @pl.when(chunk0 < nchunks)
            def _():
                pltpu.async_copy(
                    m_hbm.at[pl.ds(chunk0 * CH, CH)],
                    rows_v.at[pl.ds(b * CH, CH)], semm)

        def sub2(j, b, semm, semi, idxv):
            chunk = wid * nper + j
            buf = rows_v.at[pl.ds(b * CH, CH)]

            @pl.when(chunk < nchunks)
            def _():
                pltpu.make_async_copy(
                    m_hbm.at[pl.ds(chunk * CH, CH)], buf, semm).wait()

                @pl.when(j >= 2)
                def _():
                    pltpu.make_async_copy(
                        dst_hbm.at[pl.ds(chunk * CH, CH)], idxv, semi).wait()
                pltpu.sync_copy(buf, acc.at[idxv], add=True)

                @pl.when((j + 2 < nper) & (chunk + 2 < nchunks))
                def _():
                    pltpu.async_copy(
                        m_hbm.at[pl.ds((chunk + 2) * CH, CH)], buf, semm)
                    pltpu.async_copy(
                        dst_hbm.at[pl.ds((chunk + 2) * CH, CH)], idxv, semi)

        def step(j2, carry):
            sub2(2 * j2, 0, semm0, semi0, idx0_v)
            sub2(2 * j2 + 1, 1, semm1, semi1, idx1_v)
            return carry

        lax.fori_loop(0, nper // 2, step, 0)
        plsc.subcore_barrier()

        def drain_fire(t, carry):
            j = sid + NS * t

            @pl.when(j < ZCHUNKS)
            def _():
                r0 = j * ZROWS
                pltpu.async_copy(
                    acc.at[pl.ds(r0, ZROWS)],
                    out_hbm.at[pl.ds(cid * N + r0, ZROWS)], semz)
            return carry
        lax.fori_loop(0, nz, drain_fire, 0)

        def drain_wait(t, carry):
            j = sid + NS * t

            @pl.when(j < ZCHUNKS)
            def _():
                pltpu.make_async_copy(
                    acc.at[pl.ds(0, ZROWS)],
                    out_hbm.at[pl.ds(cid * N, ZROWS)], semz).wait()
            return carry
        lax.fori_loop(0, nz, drain_wait, 0)

    return pl.kernel(
        body,
        out_type=jax.ShapeDtypeStruct((2 * N, D), _f32),
        mesh=_sc_mesh(),
        scratch_types=[
            pltpu.VMEM_SHARED((N, D), _f32),
            pltpu.VMEM((ZROWS, D), _f32),
            pltpu.VMEM((CH,), _i32),
            pltpu.VMEM((CH,), _i32),
            pltpu.VMEM((2 * CH, D), _f32),
            pltpu.SemaphoreType.DMA,
            pltpu.SemaphoreType.DMA,
            pltpu.SemaphoreType.DMA,
            pltpu.SemaphoreType.DMA,
            pltpu.SemaphoreType.DMA,
        ],
    )


def _bdot(a, b):
    return jax.lax.dot_general(
        a, b, (((1,), (0,)), ((), ())),
        precision=jax.lax.Precision.DEFAULT,
        preferred_element_type=_f32)


def _msg_body(g1, g2, d2, w1a, w1b, w1c, b1, ga1, be1, w2, b2, ga2, be2, out):
    hd = g1[...]
    hs = g2[...]
    dist = jnp.sqrt(d2[...] + 1e-12)
    t = _bdot(hd, w1a[...]) + _bdot(hs, w1b[...]) + dist * w1c[...] + b1[...]
    t = _silu(_ln(t, ga1[...], be1[...]))
    t = _bdot(t, w2[...]) + b2[...]
    out[...] = _silu(_ln(t, ga2[...], be2[...]))


@functools.lru_cache(maxsize=None)
def _msg_call(ne):
    nbe = ne // BE
    wspec = lambda r, c: pl.BlockSpec((r, c), lambda i: (0, 0))
    return pl.pallas_call(
        _msg_body,
        grid=(nbe,),
        in_specs=[
            pl.BlockSpec((BE, D), lambda i: (i, 0)),
            pl.BlockSpec((BE, D), lambda i: (i + nbe, 0)),
            pl.BlockSpec((BE, 1), lambda i: (i, 0)),
            wspec(D, D), wspec(D, D), wspec(1, D), wspec(1, D),
            wspec(1, D), wspec(1, D),
            wspec(D, D), wspec(1, D), wspec(1, D), wspec(1, D),
        ],
        out_specs=pl.BlockSpec((BE, D), lambda i: (i, 0)),
        out_shape=jax.ShapeDtypeStruct((ne, D), _f32),
    )


def _upd_body_n(nparts):
    def body(*refs):
        hp = refs[0]
        parts = refs[1:1 + 2 * nparts]
        u1a, u1b, b1, ga1, be1, u2, b2, ga2, be2 = refs[1 + 2 * nparts:-1]
        out = refs[-1]
        h = hp[...]
        agg = parts[0][...]
        for pr in parts[1:]:
            agg = agg + pr[...]
        t = _dot(h, u1a[...]) + _dot(agg, u1b[...]) + b1[...]
        t = _silu(_ln(t, ga1[...], be1[...]))
        t = _dot(t, u2[...]) + b2[...]
        u = _silu(_ln(t, ga2[...], be2[...]))
        out[...] = h + u
    return body


@functools.lru_cache(maxsize=None)
def _upd_call(nparts):
    wspec = lambda r, c: pl.BlockSpec((r, c), lambda i: (0, 0))
    pspecs = []
    for _ in range(nparts):
        pspecs.append(pl.BlockSpec((BN, D), lambda i: (i, 0)))
        pspecs.append(pl.BlockSpec((BN, D), lambda i: (i + NBN, 0)))
    return pl.pallas_call(
        _upd_body_n(nparts),
        grid=(NBN,),
        in_specs=[pl.BlockSpec((BN, D), lambda i: (i, 0))] + pspecs + [
            wspec(D, D), wspec(D, D), wspec(1, D), wspec(1, D),
            wspec(1, D),
            wspec(D, D), wspec(1, D), wspec(1, D), wspec(1, D),
        ],
        out_specs=pl.BlockSpec((BN, D), lambda i: (i, 0)),
        out_shape=jax.ShapeDtypeStruct((N, D), _f32),
    )


def _emb_body(x, ew, eb, out):
    out[...] = x[...] * ew[...] + eb[...]


@functools.lru_cache(maxsize=None)
def _emb_call():
    return pl.pallas_call(
        _emb_body,
        out_shape=jax.ShapeDtypeStruct((N, D), _f32),
    )


def _readout_body(hp, bt, w1, b1, w2, b2, out):
    h = hp[...]
    oh = (bt[...] == lax.broadcasted_iota(_i32, (1, G), 1)).astype(_f32)
    agg = jax.lax.dot_general(
        oh, h, (((0,), (0,)), ((), ())),
        precision=jax.lax.Precision.HIGHEST,
        preferred_element_type=_f32)
    t = jnp.maximum(_dot(agg, w1[...]) + b1[...], 0.0)
    out[...] = _dot(t, w2[...]) + b2[...]


@functools.lru_cache(maxsize=None)
def _readout_call():
    return pl.pallas_call(
        _readout_body,
        out_shape=jax.ShapeDtypeStruct((G, 1), _f32),
    )


def _mlp_args(p, has_dist, wdtype=_f32):
    w1 = p['W1']
    r = lambda v: jnp.reshape(v, (1, D))
    args = [w1[:D].astype(wdtype), w1[D:2 * D].astype(wdtype)]
    if has_dist:
        args.append(w1[2 * D:2 * D + 1])
    args += [r(p['b1']), r(p['g1']), r(p['be1']),
             p['W2'].astype(wdtype), r(p['b2']), r(p['g2']), r(p['be2'])]
    return args


def kernel(x, pos, edge_index, node_subnode_index, subgraph_edge_index,
           subnode_node_index, batch, params):
    sets = [edge_index, node_subnode_index, subgraph_edge_index,
            subnode_node_index]
    idx_all = jnp.concatenate(
        [jnp.concatenate([s[1], s[0]]).astype(_i32) for s in sets])
    dsts = [[s[1][i * ES:(i + 1) * ES].astype(_i32) for i in range(NSPLIT)]
            for s in sets]
    idx2s = [[jnp.concatenate([s[1][i * ES:(i + 1) * ES],
                               s[0][i * ES:(i + 1) * ES]]).astype(_i32)
              for i in range(NSPLIT)] for s in sets]

    d2_all = _dist2_call()(pos[:, 0], pos[:, 1], pos[:, 2], idx_all)
    d2s = jnp.reshape(d2_all, (NSETS, NSPLIT, ES, 1))

    h = _emb_call()(x, jnp.reshape(params['emb_W'], (1, D)),
                    jnp.reshape(params['emb_b'], (1, D)))

    names = ['ground', 'g2s', 'sub', 's2g']
    for lp in params['layers']:
        for k, name in enumerate(names):
            p = lp[name]
            margs = _mlp_args(p['msg'], True)
            parts = []
            for i in range(NSPLIT):
                gath = _gather_call(2 * ES)(h, idx2s[k][i])
                m = _msg_call(ES)(gath, gath, d2s[k][i], *margs)
                parts.append(_scatter_call(ES)(m, dsts[k][i]))
            pargs = []
            for pa in parts:
                pargs += [pa, pa]
            h = _upd_call(NSPLIT)(h, *pargs, *_mlp_args(p['upd'], False))

    return _readout_call()(
        h, jnp.reshape(batch.astype(_i32), (N, 1)),
        params['pred_W1'], jnp.reshape(params['pred_b1'], (1, D)),
        params['pred_W2'], jnp.reshape(params['pred_b2'], (1, 1)))


# speedup vs baseline: 1.0303x; 1.0303x over previous
import functools

import jax
import jax.numpy as jnp
from jax import lax
from jax.experimental import pallas as pl
from jax.experimental.pallas import tpu as pltpu
from jax.experimental.pallas import tpu_sc as plsc

N = 10000
E = 160000
D = 128
G = 16
NSETS = 4
NC = 2
NS = 16
NW = NC * NS
CH = 128

E2 = 2 * E
NSPLIT = 2
ES = E // NSPLIT
ZROWS = 80
ZCHUNKS = N // ZROWS

DCH = 1280
DCHUNKS = NSETS * (E // DCH)
DPER = -(-DCHUNKS // NW)
DSUB = DCH // 16

BE = 2000
NBE = E // BE
BN = 2000
NBN = N // BN

_f32 = jnp.float32
_i32 = jnp.int32


def _dot(a, b):
    return jax.lax.dot_general(
        a, b, (((1,), (0,)), ((), ())),
        precision=jax.lax.Precision.HIGHEST,
        preferred_element_type=_f32)


def _ln(x, g, b):
    mu = jnp.mean(x, axis=-1, keepdims=True)
    var = jnp.mean((x - mu) * (x - mu), axis=-1, keepdims=True)
    return (x - mu) * jax.lax.rsqrt(var + 1e-5) * g + b


def _silu(x):
    return x * (0.5 * jnp.tanh(0.5 * x) + 0.5)


def _sc_mesh():
    return plsc.VectorSubcoreMesh(core_axis_name="c", subcore_axis_name="s")


def _nper(nchunks):
    n = -(-nchunks // NW)
    return n + (n % 2)


@functools.lru_cache(maxsize=None)
def _dist2_call():
    def body(px_hbm, py_hbm, pz_hbm, idx_hbm, out_hbm,
             px_v, py_v, pz_v, idxd_v, idxs_v, out_v):
        cid = lax.axis_index("c")
        sid = lax.axis_index("s")
        wid = sid * NC + cid

        pltpu.sync_copy(px_hbm, px_v)
        pltpu.sync_copy(py_hbm, py_v)
        pltpu.sync_copy(pz_hbm, pz_v)

        def step(j, carry):
            chunk = wid * DPER + j

            @pl.when(chunk < DCHUNKS)
            def _():
                s = chunk // (E // DCH)
                e0 = (chunk % (E // DCH)) * DCH
                pltpu.sync_copy(idx_hbm.at[pl.ds(s * E2 + e0, DCH)], idxd_v)
                pltpu.sync_copy(idx_hbm.at[pl.ds(s * E2 + E + e0, DCH)], idxs_v)

                def grp(k, c2):
                    o = k * 16
                    gd = idxd_v[pl.ds(o, 16)]
                    gs = idxs_v[pl.ds(o, 16)]
                    dx = (plsc.load_gather(px_v, [gd])
                          - plsc.load_gather(px_v, [gs]))
                    dy = (plsc.load_gather(py_v, [gd])
                          - plsc.load_gather(py_v, [gs]))
                    dz = (plsc.load_gather(pz_v, [gd])
                          - plsc.load_gather(pz_v, [gs]))
                    out_v[pl.ds(o, 16)] = dx * dx + dy * dy + dz * dz
                    return c2
                lax.fori_loop(0, DSUB, grp, 0)
                pltpu.sync_copy(out_v, out_hbm.at[pl.ds(chunk * DCH, DCH)])
            return carry

        lax.fori_loop(0, DPER, step, 0)

    return pl.kernel(
        body,
        out_type=jax.ShapeDtypeStruct((NSETS * E,), _f32),
        mesh=_sc_mesh(),
        scratch_types=[
            pltpu.VMEM((N,), _f32),
            pltpu.VMEM((N,), _f32),
            pltpu.VMEM((N,), _f32),
            pltpu.VMEM((DCH,), _i32),
            pltpu.VMEM((DCH,), _i32),
            pltpu.VMEM((DCH,), _f32),
        ],
        compiler_params=pltpu.CompilerParams(needs_layout_passes=False),
    )


@functools.lru_cache(maxsize=None)
def _gather_call(nidx):
    nchunks = nidx // CH
    nper = _nper(nchunks)

    def body(tbl_hbm, idx_hbm, out_hbm, idx0_v, idx1_v, rows_v,
             semg, semw0, semw1, semi0, semi1):
        cid = lax.axis_index("c")
        sid = lax.axis_index("s")
        wid = sid * NC + cid

        for b, idxv in ((0, idx0_v), (1, idx1_v)):
            chunk0 = wid * nper + b

            @pl.when(chunk0 < nchunks)
            def _():
                pltpu.sync_copy(idx_hbm.at[pl.ds(chunk0 * CH, CH)], idxv)

        def sub(j, b, semw, semi, idxv):
            chunk = wid * nper + j
            buf = rows_v.at[pl.ds(b * CH, CH)]

            @pl.when(chunk < nchunks)
            def _():
                @pl.when(j >= 2)
                def _():
                    pltpu.make_async_copy(
                        idx_hbm.at[pl.ds(chunk * CH, CH)], idxv, semi).wait()
                    pltpu.make_async_copy(
                        buf, out_hbm.at[pl.ds(chunk * CH, CH)], semw).wait()
                pltpu.async_copy(tbl_hbm.at[idxv], buf, semg).wait()
                pltpu.async_copy(buf, out_hbm.at[pl.ds(chunk * CH, CH)], semw)

                @pl.when((j + 2 < nper) & (chunk + 2 < nchunks))
                def _():
                    pltpu.async_copy(
                        idx_hbm.at[pl.ds((chunk + 2) * CH, CH)], idxv, semi)

        def step(j2, carry):
            sub(2 * j2, 0, semw0, semi0, idx0_v)
            sub(2 * j2 + 1, 1, semw1, semi1, idx1_v)
            return carry

        lax.fori_loop(0, nper // 2, step, 0)

        nvalid = jnp.maximum(jnp.minimum(nper, nchunks - wid * nper), 0)
        for b, semw in ((0, semw0), (1, semw1)):
            @pl.when(nvalid >= b + 1)
            def _():
                pltpu.make_async_copy(
                    rows_v.at[pl.ds(b * CH, CH)],
                    out_hbm.at[pl.ds(wid * nper * CH, CH)], semw).wait()

    return pl.kernel(
        body,
        out_type=jax.ShapeDtypeStruct((nidx, D), _f32),
        mesh=_sc_mesh(),
        scratch_types=[
            pltpu.VMEM((CH,), _i32),
            pltpu.VMEM((CH,), _i32),
            pltpu.VMEM((2 * CH, D), _f32),
            pltpu.SemaphoreType.DMA,
            pltpu.SemaphoreType.DMA,
            pltpu.SemaphoreType.DMA,
            pltpu.SemaphoreType.DMA,
            pltpu.SemaphoreType.DMA,
        ],
    )


@functools.lru_cache(maxsize=None)
def _scatter_call(ne):
    nchunks = ne // CH
    nper = _nper(nchunks)
    nz = -(-ZCHUNKS // NS)

    def body(m_hbm, dst_hbm, out_hbm, acc, zbuf, idx0_v, idx1_v, rows_v,
             semz, semm0, semm1, semi0, semi1):
        cid = lax.axis_index("c")
        sid = lax.axis_index("s")
        wid = sid * NC + cid

        for b, idxv in ((0, idx0_v), (1, idx1_v)):
            chunk0 = wid * nper + b

            @pl.when(chunk0 < nchunks)
            def _():
                pltpu.sync_copy(dst_hbm.at[pl.ds(chunk0 * CH, CH)], idxv)

        def zfill(i, carry):
            for j in range(8):
                zbuf[i, pl.ds(j * 16, 16)] = jnp.zeros((16,), _f32)
            return carry
        lax.fori_loop(0, ZROWS, zfill, 0)

        def zero_fire(t, carry):
            j = sid + NS * t

            @pl.when(j < ZCHUNKS)
            def _():
                pltpu.async_copy(zbuf, acc.at[pl.ds(j * ZROWS, ZROWS)], semz)
            return carry
        lax.fori_loop(0, nz, zero_fire, 0)

        def zero_drain(t, carry):
            j = sid + NS * t

            @pl.when(j < ZCHUNKS)
            def _():
                pltpu.make_async_copy(
                    zbuf, acc.at[pl.ds(0, ZROWS)], semz).wait()
            return carry
        lax.fori_loop(0, nz, zero_drain, 0)
        plsc.subcore_barrier()

        for b, semm in ((0, semm0), (1, semm1)):
            chunk0 = wid * nper + b

            @pl.when(chunk0 < nchunks)
            def _():
                pltpu.async_copy(
                    m_hbm.at[pl.ds(chunk0 * CH, CH)],
                    rows_v.at[pl.ds(b * CH, CH)], semm)

        def sub2(j, b, semm, semi, idxv):
            chunk = wid * nper + j
            buf = rows_v.at[pl.ds(b * CH, CH)]

            @pl.when(chunk < nchunks)
            def _():
                pltpu.make_async_copy(
                    m_hbm.at[pl.ds(chunk * CH, CH)], buf, semm).wait()

                @pl.when(j >= 2)
                def _():
                    pltpu.make_async_copy(
                        dst_hbm.at[pl.ds(chunk * CH, CH)], idxv, semi).wait()
                pltpu.sync_copy(buf, acc.at[idxv], add=True)

                @pl.when((j + 2 < nper) & (chunk + 2 < nchunks))
                def _():
                    pltpu.async_copy(
                        m_hbm.at[pl.ds((chunk + 2) * CH, CH)], buf, semm)
                    pltpu.async_copy(
                        dst_hbm.at[pl.ds((chunk + 2) * CH, CH)], idxv, semi)

        def step(j2, carry):
            sub2(2 * j2, 0, semm0, semi0, idx0_v)
            sub2(2 * j2 + 1, 1, semm1, semi1, idx1_v)
            return carry

        lax.fori_loop(0, nper // 2, step, 0)
        plsc.subcore_barrier()

        def drain_fire(t, carry):
            j = sid + NS * t

            @pl.when(j < ZCHUNKS)
            def _():
                r0 = j * ZROWS
                pltpu.async_copy(
                    acc.at[pl.ds(r0, ZROWS)],
                    out_hbm.at[pl.ds(cid * N + r0, ZROWS)], semz)
            return carry
        lax.fori_loop(0, nz, drain_fire, 0)

        def drain_wait(t, carry):
            j = sid + NS * t

            @pl.when(j < ZCHUNKS)
            def _():
                pltpu.make_async_copy(
                    acc.at[pl.ds(0, ZROWS)],
                    out_hbm.at[pl.ds(cid * N, ZROWS)], semz).wait()
            return carry
        lax.fori_loop(0, nz, drain_wait, 0)

    return pl.kernel(
        body,
        out_type=jax.ShapeDtypeStruct((2 * N, D), _f32),
        mesh=_sc_mesh(),
        scratch_types=[
            pltpu.VMEM_SHARED((N, D), _f32),
            pltpu.VMEM((ZROWS, D), _f32),
            pltpu.VMEM((CH,), _i32),
            pltpu.VMEM((CH,), _i32),
            pltpu.VMEM((2 * CH, D), _f32),
            pltpu.SemaphoreType.DMA,
            pltpu.SemaphoreType.DMA,
            pltpu.SemaphoreType.DMA,
            pltpu.SemaphoreType.DMA,
            pltpu.SemaphoreType.DMA,
        ],
    )


_bf16 = jnp.bfloat16


def _bdot(a, b):
    return jax.lax.dot_general(
        a.astype(_bf16), b.astype(_bf16), (((1,), (0,)), ((), ())),
        preferred_element_type=_f32)


def _msg_body(g1, g2, d2, w1a, w1b, w1c, b1, ga1, be1, w2, b2, ga2, be2, out):
    hd = g1[...]
    hs = g2[...]
    dist = jnp.sqrt(d2[...] + 1e-12)
    dterm = (dist.astype(_bf16).astype(_f32)
             * w1c[...].astype(_bf16).astype(_f32))
    t = _bdot(hd, w1a[...]) + _bdot(hs, w1b[...]) + dterm + b1[...]
    t = _silu(_ln(t, ga1[...], be1[...]))
    t = _bdot(t, w2[...]) + b2[...]
    out[...] = _silu(_ln(t, ga2[...], be2[...]))


@functools.lru_cache(maxsize=None)
def _msg_call(ne):
    nbe = ne // BE
    wspec = lambda r, c: pl.BlockSpec((r, c), lambda i: (0, 0))
    return pl.pallas_call(
        _msg_body,
        grid=(nbe,),
        in_specs=[
            pl.BlockSpec((BE, D), lambda i: (i, 0)),
            pl.BlockSpec((BE, D), lambda i: (i + nbe, 0)),
            pl.BlockSpec((BE, 1), lambda i: (i, 0)),
            wspec(D, D), wspec(D, D), wspec(1, D), wspec(1, D),
            wspec(1, D), wspec(1, D),
            wspec(D, D), wspec(1, D), wspec(1, D), wspec(1, D),
        ],
        out_specs=pl.BlockSpec((BE, D), lambda i: (i, 0)),
        out_shape=jax.ShapeDtypeStruct((ne, D), _f32),
    )


def _upd_body_n(nparts):
    def body(*refs):
        hp = refs[0]
        parts = refs[1:1 + 2 * nparts]
        u1a, u1b, b1, ga1, be1, u2, b2, ga2, be2 = refs[1 + 2 * nparts:-1]
        out = refs[-1]
        h = hp[...]
        agg = parts[0][...]
        for pr in parts[1:]:
            agg = agg + pr[...]
        t = _bdot(h, u1a[...]) + _bdot(agg, u1b[...]) + b1[...]
        t = _silu(_ln(t, ga1[...], be1[...]))
        t = _bdot(t, u2[...]) + b2[...]
        u = _silu(_ln(t, ga2[...], be2[...]))
        out[...] = h + u
    return body


@functools.lru_cache(maxsize=None)
def _upd_call(nparts):
    wspec = lambda r, c: pl.BlockSpec((r, c), lambda i: (0, 0))
    pspecs = []
    for _ in range(nparts):
        pspecs.append(pl.BlockSpec((BN, D), lambda i: (i, 0)))
        pspecs.append(pl.BlockSpec((BN, D), lambda i: (i + NBN, 0)))
    return pl.pallas_call(
        _upd_body_n(nparts),
        grid=(NBN,),
        in_specs=[pl.BlockSpec((BN, D), lambda i: (i, 0))] + pspecs + [
            wspec(D, D), wspec(D, D), wspec(1, D), wspec(1, D),
            wspec(1, D),
            wspec(D, D), wspec(1, D), wspec(1, D), wspec(1, D),
        ],
        out_specs=pl.BlockSpec((BN, D), lambda i: (i, 0)),
        out_shape=jax.ShapeDtypeStruct((N, D), _f32),
    )


def _emb_body(x, ew, eb, out):
    out[...] = x[...] * ew[...] + eb[...]


@functools.lru_cache(maxsize=None)
def _emb_call():
    return pl.pallas_call(
        _emb_body,
        out_shape=jax.ShapeDtypeStruct((N, D), _f32),
    )


def _readout_body(hp, bt, w1, b1, w2, b2, out):
    h = hp[...]
    oh = (bt[...] == lax.broadcasted_iota(_i32, (1, G), 1)).astype(_f32)
    agg = jax.lax.dot_general(
        oh, h, (((0,), (0,)), ((), ())),
        precision=jax.lax.Precision.HIGHEST,
        preferred_element_type=_f32)
    t = jnp.maximum(_bdot(agg, w1[...]) + b1[...], 0.0)
    out[...] = _bdot(t, w2[...]) + b2[...]


@functools.lru_cache(maxsize=None)
def _readout_call():
    return pl.pallas_call(
        _readout_body,
        out_shape=jax.ShapeDtypeStruct((G, 1), _f32),
    )


def _mlp_args(p, has_dist, wdtype=_f32):
    w1 = p['W1']
    r = lambda v: jnp.reshape(v, (1, D))
    args = [w1[:D].astype(wdtype), w1[D:2 * D].astype(wdtype)]
    if has_dist:
        args.append(w1[2 * D:2 * D + 1])
    args += [r(p['b1']), r(p['g1']), r(p['be1']),
             p['W2'].astype(wdtype), r(p['b2']), r(p['g2']), r(p['be2'])]
    return args


def kernel(x, pos, edge_index, node_subnode_index, subgraph_edge_index,
           subnode_node_index, batch, params):
    sets = [edge_index, node_subnode_index, subgraph_edge_index,
            subnode_node_index]
    idx_all = jnp.concatenate(
        [jnp.concatenate([s[1], s[0]]).astype(_i32) for s in sets])
    dsts = [[s[1][i * ES:(i + 1) * ES].astype(_i32) for i in range(NSPLIT)]
            for s in sets]
    idx2s = [[jnp.concatenate([s[1][i * ES:(i + 1) * ES],
                               s[0][i * ES:(i + 1) * ES]]).astype(_i32)
              for i in range(NSPLIT)] for s in sets]

    d2_all = _dist2_call()(pos[:, 0], pos[:, 1], pos[:, 2], idx_all)
    d2s = jnp.reshape(d2_all, (NSETS, NSPLIT, ES, 1))

    h = _emb_call()(x, jnp.reshape(params['emb_W'], (1, D)),
                    jnp.reshape(params['emb_b'], (1, D)))

    names = ['ground', 'g2s', 'sub', 's2g']
    for lp in params['layers']:
        for k, name in enumerate(names):
            p = lp[name]
            margs = _mlp_args(p['msg'], True)
            parts = []
            for i in range(NSPLIT):
                gath = _gather_call(2 * ES)(h, idx2s[k][i])
                m = _msg_call(ES)(gath, gath, d2s[k][i], *margs)
                parts.append(_scatter_call(ES)(m, dsts[k][i]))
            pargs = []
            for pa in parts:
                pargs += [pa, pa]
            h = _upd_call(NSPLIT)(h, *pargs, *_mlp_args(p['upd'], False))

    return _readout_call()(
        h, jnp.reshape(batch.astype(_i32), (N, 1)),
        params['pred_W1'], jnp.reshape(params['pred_b1'], (1, D)),
        params['pred_W2'], jnp.reshape(params['pred_b2'], (1, 1)))
